# baseline (speedup 1.0000x reference)
import jax
import jax.numpy as jnp
from jax import lax
from jax.experimental import pallas as pl

_C = 8
_K = 1024
_R = 512


def _rvq_kernel(x_ref, emb_ref, embt_ref, zq_ref, idx_ref):
    x = x_ref[...]
    z_q = jnp.zeros_like(x)
    x_res = x
    kio = lax.broadcasted_iota(jnp.int32, (x.shape[0], _K), 1)
    for i in range(_C):
        x_res = x_res - z_q
        emb = emb_ref[i]
        emb_t = embt_ref[i]
        xe = jnp.dot(x_res, emb_t, preferred_element_type=jnp.float32)
        x2 = jnp.sum(x_res * x_res, axis=1, keepdims=True)
        e2 = jnp.sum(emb_t * emb_t, axis=0, keepdims=True)
        dists = x2 - 2.0 * xe + e2
        dmin = jnp.min(dists, axis=1, keepdims=True)
        idx = jnp.min(jnp.where(dists == dmin, kio, _K), axis=1)
        onehot = (kio == idx[:, None]).astype(jnp.float32)
        z_q = z_q + jnp.dot(onehot, emb, precision=lax.Precision.HIGHEST,
                            preferred_element_type=jnp.float32)
        idx_ref[i, :] = idx
    zq_ref[...] = z_q


def kernel(x_in, codebooks):
    B, N, D = x_in.shape
    rows = B * N
    x = x_in.reshape(rows, D)
    embt = jnp.transpose(codebooks, (0, 2, 1))
    grid = (rows // _R,)
    zq, idx = pl.pallas_call(
        _rvq_kernel,
        grid=grid,
        in_specs=[
            pl.BlockSpec((_R, D), lambda j: (j, 0)),
            pl.BlockSpec((_C, _K, D), lambda j: (0, 0, 0)),
            pl.BlockSpec((_C, D, _K), lambda j: (0, 0, 0)),
        ],
        out_specs=[
            pl.BlockSpec((_R, D), lambda j: (j, 0)),
            pl.BlockSpec((_C, _R), lambda j: (0, j)),
        ],
        out_shape=[
            jax.ShapeDtypeStruct((rows, D), jnp.float32),
            jax.ShapeDtypeStruct((_C, rows), jnp.int32),
        ],
    )(x, codebooks, embt)
    z_q = zq.reshape(B, N, D)
    indices = idx.reshape(_C, B, N).transpose(1, 0, 2)
    return (z_q, indices)


# speedup vs baseline: 1.5415x; 1.5415x over previous
import jax
import jax.numpy as jnp
from jax import lax
from jax.experimental import pallas as pl

_C = 8
_K = 1024
_R = 512


def _rvq_kernel(x_ref, embt_ref, hi_ref, mid_ref, lo_ref, zq_ref, idx_ref):
    x = x_ref[...]
    z_q = jnp.zeros_like(x)
    x_res = x
    kio = lax.broadcasted_iota(jnp.int32, (x.shape[0], _K), 1)
    for i in range(_C):
        x_res = x_res - z_q
        emb_t = embt_ref[i]
        xe = jnp.dot(x_res, emb_t, preferred_element_type=jnp.float32)
        x2 = jnp.sum(x_res * x_res, axis=1, keepdims=True)
        e2 = jnp.sum(emb_t * emb_t, axis=0, keepdims=True)
        dists = x2 - 2.0 * xe + e2
        dmin = jnp.min(dists, axis=1, keepdims=True)
        idx = jnp.min(jnp.where(dists == dmin, kio, _K), axis=1)
        onehot = (kio == idx[:, None]).astype(jnp.bfloat16)
        g_hi = jnp.dot(onehot, hi_ref[i], preferred_element_type=jnp.float32)
        g_mid = jnp.dot(onehot, mid_ref[i], preferred_element_type=jnp.float32)
        g_lo = jnp.dot(onehot, lo_ref[i], preferred_element_type=jnp.float32)
        z_q = z_q + ((g_hi + g_mid) + g_lo)
        idx_ref[i, :] = idx
    zq_ref[...] = z_q


def kernel(x_in, codebooks):
    B, N, D = x_in.shape
    rows = B * N
    x = x_in.reshape(rows, D)
    embt = jnp.transpose(codebooks, (0, 2, 1))
    def _rp(v):
        return lax.reduce_precision(v, exponent_bits=8, mantissa_bits=7)
    h = _rp(codebooks)
    r1 = codebooks - h
    m = _rp(r1)
    r2 = r1 - m
    e_hi = h.astype(jnp.bfloat16)
    e_mid = m.astype(jnp.bfloat16)
    e_lo = _rp(r2).astype(jnp.bfloat16)
    grid = (rows // _R,)
    zq, idx = pl.pallas_call(
        _rvq_kernel,
        grid=grid,
        in_specs=[
            pl.BlockSpec((_R, D), lambda j: (j, 0)),
            pl.BlockSpec((_C, D, _K), lambda j: (0, 0, 0)),
            pl.BlockSpec((_C, _K, D), lambda j: (0, 0, 0)),
            pl.BlockSpec((_C, _K, D), lambda j: (0, 0, 0)),
            pl.BlockSpec((_C, _K, D), lambda j: (0, 0, 0)),
        ],
        out_specs=[
            pl.BlockSpec((_R, D), lambda j: (j, 0)),
            pl.BlockSpec((_C, _R), lambda j: (0, j)),
        ],
        out_shape=[
            jax.ShapeDtypeStruct((rows, D), jnp.float32),
            jax.ShapeDtypeStruct((_C, rows), jnp.int32),
        ],
    )(x, embt, e_hi, e_mid, e_lo)
    z_q = zq.reshape(B, N, D)
    indices = idx.reshape(_C, B, N).transpose(1, 0, 2)
    return (z_q, indices)
